# baseline (device time: 27243 ns/iter reference)
import jax
import jax.numpy as jnp
from jax import lax
from jax.experimental import pallas as pl
from jax.experimental.pallas import tpu as pltpu

N_DEV = 4
B = 2
SQ = 256
SKV = 512
D = 768
H_LOC = 8
DH = 64
HD_LOC = H_LOC * DH
ROWS = B * SQ
CH_SIZES = (160, 96, 144, 112)
CH_STARTS = (0, 160, 256, 400)
CH = len(CH_SIZES)
CMAX = max(CH_SIZES)
SCALE = 0.125 * 1.4426950408889634


def _body(x_ref, wq_ref, wo_ref, k_ref, v_ref, out_ref,
          sbuf, rbuf, ssem, rsem):
    my = lax.axis_index("i")
    px = 3 - my
    py = jnp.bitwise_xor(my, 1)
    pd = lax.rem(my + 2, N_DEV)

    barrier = pltpu.get_barrier_semaphore()
    for nbr in (px, py, pd):
        pl.semaphore_signal(barrier, inc=1, device_id=(nbr,),
                            device_id_type=pl.DeviceIdType.MESH)

    wq = wq_ref[...].astype(jnp.bfloat16)
    wo = wo_ref[...].astype(jnp.bfloat16)

    def exch(sph, dph, c, tgt):
        n = CH_SIZES[c]
        return pltpu.make_async_remote_copy(
            src_ref=sbuf.at[sph, c, pl.ds(0, n)],
            dst_ref=rbuf.at[dph, c, pl.ds(0, n)],
            send_sem=ssem.at[dph, c],
            recv_sem=rsem.at[dph, c],
            device_id=(tgt,),
            device_id_type=pl.DeviceIdType.MESH,
        )

    ex = [[None] * CH, [None] * CH, [None] * CH]
    part = [None] * CH
    sum_x = [None] * CH
    LAST = CH - 1

    def compute_chunk(c):
        b = c // (CH // B)
        r0, n = CH_STARTS[c], CH_SIZES[c]
        xc = x_ref[r0:r0 + n, :].astype(jnp.bfloat16)
        q = (jnp.dot(xc, wq, preferred_element_type=jnp.float32)
             * SCALE).astype(jnp.bfloat16)
        outs = []
        for h in range(H_LOC):
            qh = q[:, h * DH:(h + 1) * DH]
            kh = k_ref[b, :, h * DH:(h + 1) * DH]
            vh = v_ref[b, :, h * DH:(h + 1) * DH]
            s = lax.dot_general(qh, kh, (((1,), (1,)), ((), ())),
                                preferred_element_type=jnp.float32)
            p = jnp.exp2(s)
            l = jnp.sum(p, axis=1, keepdims=True)
            o = jnp.dot(p.astype(jnp.bfloat16), vh,
                        preferred_element_type=jnp.float32)
            outs.append((o * (1.0 / l)).astype(jnp.bfloat16))
        attn = jnp.concatenate(outs, axis=1)
        part[c] = jnp.dot(attn, wo,
                          preferred_element_type=jnp.float32)
        sbuf[0, c, :n] = part[c].astype(jnp.bfloat16)

    def start_x(c):
        ex[0][c] = exch(0, 0, c, px)
        ex[0][c].start()

    def finish_x_start_y(c):
        n = CH_SIZES[c]
        ex[0][c].wait_recv()
        sum_x[c] = part[c] + rbuf[0, c, :n].astype(jnp.float32)
        sbuf[1, c, :n] = sum_x[c].astype(jnp.bfloat16)
        ex[1][c] = exch(1, 1, c, py)
        ex[1][c].start()

    def finish_y(c):
        n = CH_SIZES[c]
        ex[1][c].wait_recv()
        total = sum_x[c] + rbuf[1, c, :n].astype(jnp.float32)
        b = c // (CH // B)
        r = CH_STARTS[c] - b * SQ
        out_ref[b, r:r + n, :] = total

    def start_a2a(c):
        for slot, tgt in ((0, px), (1, py), (2, pd)):
            ex[slot][c] = exch(0, slot, c, tgt)
            ex[slot][c].start()

    def finish_a2a(c):
        n = CH_SIZES[c]
        total = part[c]
        for slot in range(3):
            ex[slot][c].wait_recv()
            total = total + rbuf[slot, c, :n].astype(jnp.float32)
        b = c // (CH // B)
        r = CH_STARTS[c] - b * SQ
        out_ref[b, r:r + n, :] = total

    compute_chunk(0)
    pl.semaphore_wait(barrier, 3)
    start_x(0)
    compute_chunk(1)
    start_x(1)
    compute_chunk(2)
    start_x(2)
    finish_x_start_y(0)
    compute_chunk(3)
    start_a2a(3)
    finish_x_start_y(1)
    finish_x_start_y(2)
    finish_y(0)
    finish_y(1)
    finish_a2a(3)
    finish_y(2)

    for c in range(CH - 1):
        ex[0][c].wait_send()
        ex[1][c].wait_send()
    for slot in range(3):
        ex[slot][LAST].wait_send()


def kernel(x, Wq, Wo, K_ext, V_ext):
    idx = lax.axis_index("i")
    k = lax.dynamic_slice_in_dim(K_ext, idx * H_LOC, H_LOC, axis=2)
    v = lax.dynamic_slice_in_dim(V_ext, idx * H_LOC, H_LOC, axis=2)
    kb = k.reshape(B, SKV, HD_LOC).astype(jnp.bfloat16)
    vb = v.reshape(B, SKV, HD_LOC).astype(jnp.bfloat16)
    out = pl.pallas_call(
        _body,
        out_shape=jax.ShapeDtypeStruct((B, SQ, D), jnp.float32),
        in_specs=[pl.BlockSpec(memory_space=pltpu.VMEM)] * 5,
        out_specs=pl.BlockSpec(memory_space=pltpu.VMEM),
        scratch_shapes=[
            pltpu.VMEM((2, CH, CMAX, D), jnp.bfloat16),
            pltpu.VMEM((3, CH, CMAX, D), jnp.bfloat16),
            pltpu.SemaphoreType.DMA((3, CH)),
            pltpu.SemaphoreType.DMA((3, CH)),
        ],
        compiler_params=pltpu.CompilerParams(collective_id=0),
    )(x.reshape(ROWS, D), Wq, Wo, kb, vb)
    return out


# device time: 25465 ns/iter; 1.0698x vs baseline; 1.0698x over previous
import jax
import jax.numpy as jnp
from jax import lax
from jax.experimental import pallas as pl
from jax.experimental.pallas import tpu as pltpu

N_DEV = 4
B = 2
SQ = 256
SKV = 512
D = 768
H_LOC = 8
DH = 64
HD_LOC = H_LOC * DH
ROWS = B * SQ
CH_SIZES = (160, 96, 160, 96)
CH_STARTS = (0, 160, 256, 416)
CH = len(CH_SIZES)
CMAX = max(CH_SIZES)
SCALE = 0.125 * 1.4426950408889634


def _body(x_ref, wq_ref, wo_ref, k_ref, v_ref, out_ref,
          sbuf, rbuf, ssem, rsem):
    my = lax.axis_index("i")
    px = 3 - my
    py = jnp.bitwise_xor(my, 1)

    barrier = pltpu.get_barrier_semaphore()
    for nbr in (px, py):
        pl.semaphore_signal(barrier, inc=1, device_id=(nbr,),
                            device_id_type=pl.DeviceIdType.MESH)

    wq = wq_ref[...].astype(jnp.bfloat16)
    wo = wo_ref[...].astype(jnp.bfloat16)

    def exch(ph, c, tgt):
        n = CH_SIZES[c]
        return pltpu.make_async_remote_copy(
            src_ref=sbuf.at[ph, c, pl.ds(0, n)],
            dst_ref=rbuf.at[ph, c, pl.ds(0, n)],
            send_sem=ssem.at[ph, c],
            recv_sem=rsem.at[ph, c],
            device_id=(tgt,),
            device_id_type=pl.DeviceIdType.MESH,
        )

    ex = [[None] * CH, [None] * CH]
    part = [None] * CH
    sum_x = [None] * CH

    def compute_chunk(c):
        b = c // (CH // B)
        r0, n = CH_STARTS[c], CH_SIZES[c]
        xc = x_ref[r0:r0 + n, :].astype(jnp.bfloat16)
        q = (jnp.dot(xc, wq, preferred_element_type=jnp.float32)
             * SCALE).astype(jnp.bfloat16)
        outs = []
        for h in range(H_LOC):
            qh = q[:, h * DH:(h + 1) * DH]
            kh = k_ref[b, :, h * DH:(h + 1) * DH]
            vh = v_ref[b, :, h * DH:(h + 1) * DH]
            s = lax.dot_general(qh, kh, (((1,), (1,)), ((), ())),
                                preferred_element_type=jnp.float32)
            p = jnp.exp2(s)
            l = jnp.sum(p, axis=1, keepdims=True)
            o = jnp.dot(p.astype(jnp.bfloat16), vh,
                        preferred_element_type=jnp.float32)
            outs.append((o * (1.0 / l)).astype(jnp.bfloat16))
        attn = jnp.concatenate(outs, axis=1)
        part[c] = jnp.dot(attn, wo,
                          preferred_element_type=jnp.float32)
        sbuf[0, c, :n] = part[c].astype(jnp.bfloat16)

    def start_x(c):
        ex[0][c] = exch(0, c, px)
        ex[0][c].start()

    def finish_x_start_y(c):
        n = CH_SIZES[c]
        ex[0][c].wait_recv()
        sum_x[c] = part[c] + rbuf[0, c, :n].astype(jnp.float32)
        sbuf[1, c, :n] = sum_x[c].astype(jnp.bfloat16)
        ex[1][c] = exch(1, c, py)
        ex[1][c].start()

    def finish_y(c):
        n = CH_SIZES[c]
        ex[1][c].wait_recv()
        total = sum_x[c] + rbuf[1, c, :n].astype(jnp.float32)
        b = c // (CH // B)
        r = CH_STARTS[c] - b * SQ
        out_ref[b, r:r + n, :] = total

    compute_chunk(0)
    pl.semaphore_wait(barrier, 2)
    start_x(0)
    compute_chunk(1)
    start_x(1)
    compute_chunk(2)
    start_x(2)
    finish_x_start_y(0)
    compute_chunk(3)
    start_x(3)
    finish_x_start_y(1)
    finish_x_start_y(2)
    finish_x_start_y(3)
    finish_y(0)
    finish_y(1)
    finish_y(2)
    finish_y(3)

    for ph in range(2):
        for c in range(CH):
            ex[ph][c].wait_send()


def kernel(x, Wq, Wo, K_ext, V_ext):
    idx = lax.axis_index("i")
    k = lax.dynamic_slice_in_dim(K_ext, idx * H_LOC, H_LOC, axis=2)
    v = lax.dynamic_slice_in_dim(V_ext, idx * H_LOC, H_LOC, axis=2)
    kb = k.reshape(B, SKV, HD_LOC).astype(jnp.bfloat16)
    vb = v.reshape(B, SKV, HD_LOC).astype(jnp.bfloat16)
    out = pl.pallas_call(
        _body,
        out_shape=jax.ShapeDtypeStruct((B, SQ, D), jnp.float32),
        in_specs=[pl.BlockSpec(memory_space=pltpu.VMEM)] * 5,
        out_specs=pl.BlockSpec(memory_space=pltpu.VMEM),
        scratch_shapes=[
            pltpu.VMEM((2, CH, CMAX, D), jnp.bfloat16),
            pltpu.VMEM((2, CH, CMAX, D), jnp.bfloat16),
            pltpu.SemaphoreType.DMA((2, CH)),
            pltpu.SemaphoreType.DMA((2, CH)),
        ],
        compiler_params=pltpu.CompilerParams(collective_id=0),
    )(x.reshape(ROWS, D), Wq, Wo, kb, vb)
    return out
